# baseline (speedup 1.0000x reference)
import functools

import jax
import jax.numpy as jnp
from jax.experimental import pallas as pl
from jax.experimental.pallas import tpu as pltpu

_TILE = 2048
_NBUF = 4
_R = 4


def _fc_kernel(nfull, tail, x_ref, w_ref, b_ref, o_ref, acc_ref, tacc_ref,
               sems, tsem):
    j = pl.program_id(0)
    nstep = pl.num_programs(0)
    slot = jax.lax.rem(j, _NBUF)
    batch = acc_ref.shape[1]
    rb = batch // _R

    def full_copy(step, s):
        col = pl.multiple_of(step * _TILE, _TILE)
        return [
            pltpu.make_async_copy(
                acc_ref.at[s, pl.ds(r * rb, rb), :],
                o_ref.at[pl.ds(r * rb, rb), pl.ds(col, _TILE)],
                sems.at[s, r],
            )
            for r in range(_R)
        ]

    def tail_copy():
        return pltpu.make_async_copy(
            tacc_ref,
            o_ref.at[:, pl.ds(nfull * _TILE, tail)],
            tsem,
        )

    @pl.when(j >= _NBUF)
    def _wait_prev():
        for c in full_copy(j - _NBUF, slot):
            c.wait()

    xb = x_ref[...].astype(jnp.bfloat16)
    wb = w_ref[...].astype(jnp.bfloat16)
    acc = jax.lax.dot_general(
        xb, wb, (((1,), (1,)), ((), ())),
        preferred_element_type=jnp.float32,
    ) + b_ref[...]

    @pl.when(j < nfull)
    def _start_full():
        acc_ref[slot] = acc
        for c in full_copy(j, slot):
            c.start()

    if tail:
        @pl.when(j == nfull)
        def _start_tail():
            tacc_ref[...] = acc[:, :tail]
            tail_copy().start()

    @pl.when(j == nstep - 1)
    def _drain():
        for d in range(1 if tail else 0, _NBUF):
            pj = j - d
            for c in full_copy(pj, jax.lax.rem(pj, _NBUF)):
                c.wait()
        if tail:
            tail_copy().wait()


def kernel(x, W, b):
    batch, in_dim = x.shape
    out_dim = W.shape[0]
    nfull = out_dim // _TILE
    tail = out_dim - nfull * _TILE
    nstep = nfull + (1 if tail else 0)
    b2 = b.reshape(1, out_dim)
    return pl.pallas_call(
        functools.partial(_fc_kernel, nfull, tail),
        grid=(nstep,),
        in_specs=[
            pl.BlockSpec((batch, in_dim), lambda j: (0, 0)),
            pl.BlockSpec((_TILE, in_dim), lambda j: (j, 0)),
            pl.BlockSpec((1, _TILE), lambda j: (0, j)),
        ],
        out_specs=pl.BlockSpec(memory_space=pl.ANY),
        out_shape=jax.ShapeDtypeStruct((batch, out_dim), jnp.float32),
        scratch_shapes=[
            pltpu.VMEM((_NBUF, batch, _TILE), jnp.float32),
            pltpu.VMEM((batch, tail if tail else 128), jnp.float32),
            pltpu.SemaphoreType.DMA((_NBUF, _R)),
            pltpu.SemaphoreType.DMA,
        ],
        compiler_params=pltpu.CompilerParams(
            dimension_semantics=("arbitrary",),
        ),
    )(x, W, b2)
